# baseline (speedup 1.0000x reference)
import functools

import jax
import jax.numpy as jnp
from jax import lax
from jax.experimental import pallas as pl
from jax.experimental.pallas import tpu as pltpu
from jax.experimental.pallas import tpu_sc as plsc

NC, NS, LANES = 2, 16, 16
NW = NC * NS

B, L, D = 1024, 200, 128
N = B * L
NTOK_PER_W = N // NW
CHUNK = 128
NCHUNK = NTOK_PER_W // CHUNK
NGROUP = CHUNK // LANES

MAX_DEPTH, MAX_SIBLING, N_NODE_TYPES = 64, 256, 4

_INV_D = 1.0 / D
_EPS = 1e-5


def _rsqrt(v):
    bits = plsc.bitcast(v, jnp.int32)
    seed = jnp.full((LANES,), 0x5F3759DF, jnp.int32) - (bits >> 1)
    y = plsc.bitcast(seed, jnp.float32)
    half_v = v * 0.5
    for _ in range(3):
        y = y * (1.5 - half_v * y * y)
    return y


NJ = D // LANES
UNROLL = 8


def _body(tok_hbm, nt_hbm, dep_hbm, sib_hbm, par_hbm,
          key_hbm, val_hbm, dept_hbm, sibt_hbm, ntt_hbm, part_hbm,
          gam_hbm, bet_hbm, out_hbm,
          tok_i, nt_i, dep_i, sib_i, par_i,
          keyrows, valrows, parrows, xbuf, isk_f,
          dep_t, sib_t, nt_t, gam_v, bet_v, sem):
    wid = lax.axis_index("s") * NC + lax.axis_index("c")
    wbase = wid * NTOK_PER_W

    pltpu.sync_copy(dept_hbm, dep_t)
    pltpu.sync_copy(sibt_hbm, sib_t)
    pltpu.sync_copy(ntt_hbm, nt_t)
    pltpu.sync_copy(gam_hbm, gam_v)
    pltpu.sync_copy(bet_hbm, bet_v)

    iota = lax.iota(jnp.int32, LANES)
    colv = [iota + (LANES * j) for j in range(NJ)]
    gms = [gam_v[pl.ds(LANES * j, LANES)] for j in range(NJ)]
    bts = [bet_v[pl.ds(LANES * j, LANES)] for j in range(NJ)]

    def chunk_body(c, carry):
        base = wbase + c * CHUNK
        sl = pl.ds(base, CHUNK)
        pltpu.sync_copy(tok_hbm.at[sl], tok_i)
        pltpu.sync_copy(nt_hbm.at[sl], nt_i)
        pltpu.sync_copy(dep_hbm.at[sl], dep_i)
        pltpu.sync_copy(sib_hbm.at[sl], sib_i)
        pltpu.sync_copy(par_hbm.at[sl], par_i)

        d1 = pltpu.async_copy(key_hbm.at[tok_i], keyrows, sem)
        d2 = pltpu.async_copy(val_hbm.at[tok_i], valrows, sem)
        d3 = pltpu.async_copy(part_hbm.at[par_i], parrows, sem)

        for g in range(NGROUP):
            gsl = pl.ds(g * LANES, LANES)
            ntg = nt_i[gsl]
            isk_f[gsl] = jnp.where((ntg == 0) | (ntg == 2), 1.0, 0.0)

        d1.wait()
        d2.wait()
        d3.wait()

        @plsc.parallel_loop(0, CHUNK, 1, unroll=UNROLL)
        def token_body(t):
            tsp = jnp.full((LANES,), t, jnp.int32)
            m = plsc.load_gather(isk_f, [tsp])
            dsp = plsc.load_gather(dep_i, [tsp])
            ssp = plsc.load_gather(sib_i, [tsp])
            nsp = plsc.load_gather(nt_i, [tsp])
            xs = []
            s = jnp.zeros((LANES,), jnp.float32)
            ss = jnp.zeros((LANES,), jnp.float32)
            for j in range(NJ):
                kd = plsc.load_gather(keyrows, [tsp, colv[j]])
                vd = plsc.load_gather(valrows, [tsp, colv[j]])
                pd = plsc.load_gather(parrows, [tsp, colv[j]])
                dd = plsc.load_gather(dep_t, [dsp, colv[j]])
                sd = plsc.load_gather(sib_t, [ssp, colv[j]])
                nd = plsc.load_gather(nt_t, [nsp, colv[j]])
                x = vd + (kd - vd) * m + pd + dd + sd + nd
                xs.append(x)
                s = s + x
                ss = ss + x * x
            mu = jnp.full((LANES,), jnp.sum(s)) * _INV_D
            msq = jnp.full((LANES,), jnp.sum(ss)) * _INV_D
            a = _rsqrt(msq - mu * mu + _EPS)
            b = -mu * a
            for j in range(NJ):
                y = (xs[j] * a + b) * gms[j] + bts[j]
                plsc.store_scatter(xbuf, [tsp, colv[j]], y)

        pltpu.sync_copy(xbuf, out_hbm.at[sl])
        return carry

    lax.fori_loop(0, NCHUNK, chunk_body, 0)


@jax.jit
def _sc_embed(tok, nt, dep, sib, par, key_table, value_table, depth_table,
              sibling_table, node_type_table, parent_key_table,
              ln_gamma, ln_beta):
    mesh = plsc.VectorSubcoreMesh(core_axis_name="c", subcore_axis_name="s",
                                  num_cores=NC, num_subcores=NS)
    fn = pl.kernel(
        _body,
        out_type=jax.ShapeDtypeStruct((N, D), jnp.float32),
        mesh=mesh,
        compiler_params=pltpu.CompilerParams(needs_layout_passes=False),
        scratch_types=[
            pltpu.VMEM((CHUNK,), jnp.int32),
            pltpu.VMEM((CHUNK,), jnp.int32),
            pltpu.VMEM((CHUNK,), jnp.int32),
            pltpu.VMEM((CHUNK,), jnp.int32),
            pltpu.VMEM((CHUNK,), jnp.int32),
            pltpu.VMEM((CHUNK, D), jnp.float32),
            pltpu.VMEM((CHUNK, D), jnp.float32),
            pltpu.VMEM((CHUNK, D), jnp.float32),
            pltpu.VMEM((CHUNK, D), jnp.float32),
            pltpu.VMEM((CHUNK,), jnp.float32),
            pltpu.VMEM((MAX_DEPTH, D), jnp.float32),
            pltpu.VMEM((MAX_SIBLING, D), jnp.float32),
            pltpu.VMEM((N_NODE_TYPES, D), jnp.float32),
            pltpu.VMEM((D,), jnp.float32),
            pltpu.VMEM((D,), jnp.float32),
            pltpu.SemaphoreType.DMA,
        ],
    )
    return fn(tok, nt, dep, sib, par, key_table, value_table, depth_table,
              sibling_table, node_type_table, parent_key_table,
              ln_gamma, ln_beta)


def kernel(token_ids, node_types, depths, sibling_indices, parent_key_ids,
           key_table, value_table, depth_table, sibling_table,
           node_type_table, parent_key_table, ln_gamma, ln_beta):
    tok = token_ids.reshape(N).astype(jnp.int32)
    nt = node_types.reshape(N).astype(jnp.int32)
    dep = depths.reshape(N).astype(jnp.int32)
    sib = sibling_indices.reshape(N).astype(jnp.int32)
    par = parent_key_ids.reshape(N).astype(jnp.int32)
    out = _sc_embed(tok, nt, dep, sib, par,
                    key_table, value_table, depth_table, sibling_table,
                    node_type_table, parent_key_table,
                    ln_gamma.astype(jnp.float32), ln_beta.astype(jnp.float32))
    return out.reshape(B, L, D)


# speedup vs baseline: 1.0954x; 1.0954x over previous
import functools

import jax
import jax.numpy as jnp
from jax import lax
from jax.experimental import pallas as pl
from jax.experimental.pallas import tpu as pltpu
from jax.experimental.pallas import tpu_sc as plsc

NC, NS, LANES = 2, 16, 16
NW = NC * NS

B, L, D = 1024, 200, 128
N = B * L
NTOK_PER_W = N // NW
CHUNK = 64
SUPER = 4
SUPTOK = CHUNK * SUPER
NSUPER = NTOK_PER_W // SUPTOK

MAX_DEPTH, MAX_SIBLING, N_NODE_TYPES = 64, 256, 4

_INV_D = 1.0 / D
_EPS = 1e-5


def _rsqrt(v):
    bits = plsc.bitcast(v, jnp.int32)
    seed = jnp.full((LANES,), 0x5F3759DF, jnp.int32) - (bits >> 1)
    y = plsc.bitcast(seed, jnp.float32)
    half_v = v * 0.5
    for _ in range(3):
        y = y * (1.5 - half_v * y * y)
    return y


NJ = D // LANES
UNROLL = 4


def _body(tok_hbm, nt_hbm, dep_hbm, sib_hbm, par_hbm,
          key_hbm, val_hbm, dept_hbm, sibt_hbm, ntt_hbm, part_hbm,
          gam_hbm, bet_hbm, out_hbm,
          tok_i, nt_i, dep_i, sib_i, par_i, isk_f,
          keyr0, valr0, parr0, keyr1, valr1, parr1, xbuf0, xbuf1,
          dep_t, sib_t, nt_t, gam_v, bet_v,
          sem_i, sem_g0, sem_g1, sem_o0, sem_o1):
    wid = lax.axis_index("s") * NC + lax.axis_index("c")
    wbase = wid * NTOK_PER_W

    pltpu.sync_copy(dept_hbm, dep_t)
    pltpu.sync_copy(sibt_hbm, sib_t)
    pltpu.sync_copy(ntt_hbm, nt_t)
    pltpu.sync_copy(gam_hbm, gam_v)
    pltpu.sync_copy(bet_hbm, bet_v)

    iota = lax.iota(jnp.int32, LANES)
    colv = [iota + (LANES * j) for j in range(NJ)]
    gms = [gam_v[pl.ds(LANES * j, LANES)] for j in range(NJ)]
    bts = [bet_v[pl.ds(LANES * j, LANES)] for j in range(NJ)]

    rows = [(keyr0, valr0, parr0, sem_g0), (keyr1, valr1, parr1, sem_g1)]
    xbufs = [(xbuf0, sem_o0), (xbuf1, sem_o1)]
    idx_bufs = (tok_i, nt_i, dep_i, sib_i, par_i)
    idx_srcs = (tok_hbm, nt_hbm, dep_hbm, sib_hbm, par_hbm)

    def issue_idx(dst_half, src_base):
        src_base = pl.multiple_of(src_base, 64)
        return [pltpu.async_copy(src.at[pl.ds(src_base, SUPTOK)],
                                 buf.at[dst_half], sem_i)
                for src, buf in zip(idx_srcs, idx_bufs)]

    def issue_gathers(half, idx_off, rset):
        kb, vb, pb, sg = rows[rset]
        tsl = tok_i.at[half, pl.ds(idx_off, CHUNK)]
        psl = par_i.at[half, pl.ds(idx_off, CHUNK)]
        return [pltpu.async_copy(key_hbm.at[tsl], kb, sg),
                pltpu.async_copy(val_hbm.at[tsl], vb, sg),
                pltpu.async_copy(part_hbm.at[psl], pb, sg)]

    def drain_gathers(rset):
        kb, vb, pb, sg = rows[rset]
        pltpu.make_async_copy(key_hbm.at[pl.ds(0, CHUNK)], kb, sg).wait()
        pltpu.make_async_copy(val_hbm.at[pl.ds(0, CHUNK)], vb, sg).wait()
        pltpu.make_async_copy(part_hbm.at[pl.ds(0, CHUNK)], pb, sg).wait()

    def drain_out(xset):
        xb, so = xbufs[xset]
        pltpu.make_async_copy(xb, out_hbm.at[pl.ds(wbase, CHUNK)], so).wait()

    pltpu.async_copy(
        xbuf0, out_hbm.at[pl.ds(wbase + NTOK_PER_W - 2 * CHUNK, CHUNK)],
        sem_o0)
    pltpu.async_copy(
        xbuf1, out_hbm.at[pl.ds(wbase + NTOK_PER_W - CHUNK, CHUNK)],
        sem_o1)
    for dsc in issue_idx(0, wbase):
        dsc.wait()
    issue_gathers(0, 0, 0)

    def super_body(s, carry):
        p = lax.rem(s, 2)
        pn = 1 - p
        psp = jnp.full((LANES,), p, jnp.int32)

        nbase = wbase + (s + 1) * SUPTOK
        idx_descs = issue_idx(pn, nbase)

        for g in range(SUPTOK // LANES):
            ntg = plsc.load_gather(nt_i, [psp, iota + g * LANES])
            isk_f[pl.ds(g * LANES, LANES)] = jnp.where(
                (ntg == 0) | (ntg == 2), 1.0, 0.0)

        for ch in range(SUPER):
            rset = ch % 2
            if ch < SUPER - 1:
                issue_gathers(p, (ch + 1) * CHUNK, 1 - rset)
            else:
                for dsc in idx_descs:
                    dsc.wait()
                issue_gathers(pn, 0, 0)

            drain_gathers(rset)
            drain_out(rset)

            kb, vb, pb, _sg = rows[rset]
            xb, so = xbufs[rset]
            loc_off = ch * CHUNK

            @plsc.parallel_loop(0, CHUNK, 1, unroll=UNROLL)
            def token_body(t):
                tsp = jnp.full((LANES,), t, jnp.int32)
                tsk = jnp.full((LANES,), loc_off + t, jnp.int32)
                m = plsc.load_gather(isk_f, [tsk])
                dsp = plsc.load_gather(dep_i, [psp, tsk])
                ssp = plsc.load_gather(sib_i, [psp, tsk])
                nsp = plsc.load_gather(nt_i, [psp, tsk])
                xs = []
                acs = jnp.zeros((LANES,), jnp.float32)
                acq = jnp.zeros((LANES,), jnp.float32)
                for j in range(NJ):
                    kd = plsc.load_gather(kb, [tsp, colv[j]])
                    vd = plsc.load_gather(vb, [tsp, colv[j]])
                    pd = plsc.load_gather(pb, [tsp, colv[j]])
                    dd = plsc.load_gather(dep_t, [dsp, colv[j]])
                    sd = plsc.load_gather(sib_t, [ssp, colv[j]])
                    nd = plsc.load_gather(nt_t, [nsp, colv[j]])
                    x = vd + (kd - vd) * m + pd + dd + sd + nd
                    xs.append(x)
                    acs = acs + x
                    acq = acq + x * x
                mu = jnp.full((LANES,), jnp.sum(acs)) * _INV_D
                msq = jnp.full((LANES,), jnp.sum(acq)) * _INV_D
                a = _rsqrt(msq - mu * mu + _EPS)
                b = -mu * a
                for j in range(NJ):
                    y = (xs[j] * a + b) * gms[j] + bts[j]
                    plsc.store_scatter(xb, [tsp, colv[j]], y)

            pltpu.async_copy(
                xb,
                out_hbm.at[pl.ds(pl.multiple_of(wbase + s * SUPTOK + loc_off, 64), CHUNK)],
                so)
        return carry

    lax.fori_loop(0, NSUPER, super_body, 0)

    drain_gathers(0)
    drain_out(0)
    drain_out(1)


@jax.jit
def _sc_embed(tok, nt, dep, sib, par, key_table, value_table, depth_table,
              sibling_table, node_type_table, parent_key_table,
              ln_gamma, ln_beta):
    mesh = plsc.VectorSubcoreMesh(core_axis_name="c", subcore_axis_name="s",
                                  num_cores=NC, num_subcores=NS)
    fn = pl.kernel(
        _body,
        out_type=jax.ShapeDtypeStruct((N, D), jnp.float32),
        mesh=mesh,
        compiler_params=pltpu.CompilerParams(needs_layout_passes=False),
        scratch_types=(
            [pltpu.VMEM((2, SUPTOK), jnp.int32)] * 5
            + [pltpu.VMEM((SUPTOK,), jnp.float32)]
            + [pltpu.VMEM((CHUNK, D), jnp.float32)] * 8
            + [pltpu.VMEM((MAX_DEPTH, D), jnp.float32),
               pltpu.VMEM((MAX_SIBLING, D), jnp.float32),
               pltpu.VMEM((N_NODE_TYPES, D), jnp.float32),
               pltpu.VMEM((D,), jnp.float32),
               pltpu.VMEM((D,), jnp.float32)]
            + [pltpu.SemaphoreType.DMA] * 5
        ),
    )
    return fn(tok, nt, dep, sib, par, key_table, value_table, depth_table,
              sibling_table, node_type_table, parent_key_table,
              ln_gamma, ln_beta)


def kernel(token_ids, node_types, depths, sibling_indices, parent_key_ids,
           key_table, value_table, depth_table, sibling_table,
           node_type_table, parent_key_table, ln_gamma, ln_beta):
    pad = [(0, SUPTOK)]
    tok = jnp.pad(token_ids.reshape(N).astype(jnp.int32), pad)
    nt = jnp.pad(node_types.reshape(N).astype(jnp.int32), pad)
    dep = jnp.pad(depths.reshape(N).astype(jnp.int32), pad)
    sib = jnp.pad(sibling_indices.reshape(N).astype(jnp.int32), pad)
    par = jnp.pad(parent_key_ids.reshape(N).astype(jnp.int32), pad)
    out = _sc_embed(tok, nt, dep, sib, par,
                    key_table, value_table, depth_table, sibling_table,
                    node_type_table, parent_key_table,
                    ln_gamma.astype(jnp.float32), ln_beta.astype(jnp.float32))
    return out.reshape(B, L, D)


# speedup vs baseline: 1.5629x; 1.4267x over previous
import jax
import jax.numpy as jnp
from jax import lax
from jax.experimental import pallas as pl
from jax.experimental.pallas import tpu as pltpu
from jax.experimental.pallas import tpu_sc as plsc

NC, NS, LANES = 2, 16, 16
NW = NC * NS

B, L, D = 1024, 200, 128
N = B * L
NTOK_PER_W = N // NW
CHUNK = 128
NCHUNK = NTOK_PER_W // CHUNK
NPAIR = NCHUNK // 2
NGROUP = CHUNK // LANES
NJ = D // LANES
UNROLL = 4

MAX_DEPTH, MAX_SIBLING, N_NODE_TYPES = 64, 256, 4

_INV_D = 1.0 / D
_EPS = 1e-5


def _rsqrt(v):
    bits = plsc.bitcast(v, jnp.int32)
    seed = jnp.full((LANES,), 0x5F3759DF, jnp.int32) - (bits >> 1)
    y = plsc.bitcast(seed, jnp.float32)
    half_v = v * 0.5
    for _ in range(3):
        y = y * (1.5 - half_v * y * y)
    return y


def _body(tok_hbm, nt_hbm, dep_hbm, sib_hbm, par_hbm,
          key_hbm, val_hbm, dept_hbm, sibt_hbm, ntt_hbm, part_hbm,
          gam_hbm, bet_hbm, out_hbm,
          tok_a, nt_a, dep_a, sib_a, par_a, isk_a,
          tok_b, nt_b, dep_b, sib_b, par_b, isk_b,
          keyrows, valrows, parrows, xbuf0, xbuf1,
          dep_t, sib_t, nt_t, gam_v, bet_v,
          sem_ia, sem_ib, sem_g, sem_o0, sem_o1):
    wid = lax.axis_index("s") * NC + lax.axis_index("c")
    wbase = wid * NTOK_PER_W

    pltpu.sync_copy(dept_hbm, dep_t)
    pltpu.sync_copy(sibt_hbm, sib_t)
    pltpu.sync_copy(ntt_hbm, nt_t)
    pltpu.sync_copy(gam_hbm, gam_v)
    pltpu.sync_copy(bet_hbm, bet_v)

    iota = lax.iota(jnp.int32, LANES)
    colv = [iota + (LANES * j) for j in range(NJ)]
    gms = [gam_v[pl.ds(LANES * j, LANES)] for j in range(NJ)]
    bts = [bet_v[pl.ds(LANES * j, LANES)] for j in range(NJ)]

    idx_srcs = (tok_hbm, nt_hbm, dep_hbm, sib_hbm, par_hbm)
    idx_a = (tok_a, nt_a, dep_a, sib_a, par_a)
    idx_b = (tok_b, nt_b, dep_b, sib_b, par_b)

    def issue_idx(bufs, sem, base):
        return [pltpu.async_copy(src.at[pl.ds(base, CHUNK)], buf, sem)
                for src, buf in zip(idx_srcs, bufs)]

    def drain_idx(bufs, sem):
        for src, buf in zip(idx_srcs, bufs):
            pltpu.make_async_copy(src.at[pl.ds(0, CHUNK)], buf, sem).wait()

    def issue_gathers(tok_i, par_i):
        return [pltpu.async_copy(key_hbm.at[tok_i], keyrows, sem_g),
                pltpu.async_copy(val_hbm.at[tok_i], valrows, sem_g),
                pltpu.async_copy(part_hbm.at[par_i], parrows, sem_g)]

    def build_mask(nt_i, isk_f):
        for g in range(NGROUP):
            gsl = pl.ds(g * LANES, LANES)
            ntg = nt_i[gsl]
            isk_f[gsl] = jnp.where((ntg == 0) | (ntg == 2), 1.0, 0.0)

    def compute(dep_i, sib_i, nt_i, isk_f, xb):
        @plsc.parallel_loop(0, CHUNK, 1, unroll=UNROLL)
        def token_body(t):
            tsp = jnp.full((LANES,), t, jnp.int32)
            m = plsc.load_gather(isk_f, [tsp])
            dsp = plsc.load_gather(dep_i, [tsp])
            ssp = plsc.load_gather(sib_i, [tsp])
            nsp = plsc.load_gather(nt_i, [tsp])
            xs = []
            acs = jnp.zeros((LANES,), jnp.float32)
            acq = jnp.zeros((LANES,), jnp.float32)
            for j in range(NJ):
                kd = plsc.load_gather(keyrows, [tsp, colv[j]])
                vd = plsc.load_gather(valrows, [tsp, colv[j]])
                pd = plsc.load_gather(parrows, [tsp, colv[j]])
                dd = plsc.load_gather(dep_t, [dsp, colv[j]])
                sd = plsc.load_gather(sib_t, [ssp, colv[j]])
                nd = plsc.load_gather(nt_t, [nsp, colv[j]])
                x = vd + (kd - vd) * m + pd + dd + sd + nd
                xs.append(x)
                acs = acs + x
                acq = acq + x * x
            mu = jnp.full((LANES,), jnp.sum(acs)) * _INV_D
            msq = jnp.full((LANES,), jnp.sum(acq)) * _INV_D
            a = _rsqrt(msq - mu * mu + _EPS)
            b = -mu * a
            for j in range(NJ):
                y = (xs[j] * a + b) * gms[j] + bts[j]
                plsc.store_scatter(xb, [tsp, colv[j]], y)

    def drain_out(xb, so):
        pltpu.make_async_copy(xb, out_hbm.at[pl.ds(wbase, CHUNK)], so).wait()

    pltpu.async_copy(
        xbuf0, out_hbm.at[pl.ds(wbase + NTOK_PER_W - 2 * CHUNK, CHUNK)],
        sem_o0)
    pltpu.async_copy(
        xbuf1, out_hbm.at[pl.ds(wbase + NTOK_PER_W - CHUNK, CHUNK)],
        sem_o1)
    issue_idx(idx_a, sem_ia, wbase)

    def pair_body(bp, carry):
        base_a = wbase + bp * (2 * CHUNK)
        base_b = base_a + CHUNK

        drain_idx(idx_a, sem_ia)
        ga = issue_gathers(tok_a, par_a)
        ib = issue_idx(idx_b, sem_ib, base_b)
        build_mask(nt_a, isk_a)
        for dsc in ga:
            dsc.wait()
        drain_out(xbuf0, sem_o0)
        compute(dep_a, sib_a, nt_a, isk_a, xbuf0)
        pltpu.async_copy(xbuf0, out_hbm.at[pl.ds(base_a, CHUNK)], sem_o0)

        for dsc in ib:
            dsc.wait()
        gb = issue_gathers(tok_b, par_b)
        issue_idx(idx_a, sem_ia, base_b + CHUNK)
        build_mask(nt_b, isk_b)
        for dsc in gb:
            dsc.wait()
        drain_out(xbuf1, sem_o1)
        compute(dep_b, sib_b, nt_b, isk_b, xbuf1)
        pltpu.async_copy(xbuf1, out_hbm.at[pl.ds(base_b, CHUNK)], sem_o1)
        return carry

    lax.fori_loop(0, NPAIR, pair_body, 0)

    drain_idx(idx_a, sem_ia)
    drain_out(xbuf0, sem_o0)
    drain_out(xbuf1, sem_o1)


@jax.jit
def _sc_embed(tok, nt, dep, sib, par, key_table, value_table, depth_table,
              sibling_table, node_type_table, parent_key_table,
              ln_gamma, ln_beta):
    mesh = plsc.VectorSubcoreMesh(core_axis_name="c", subcore_axis_name="s",
                                  num_cores=NC, num_subcores=NS)
    fn = pl.kernel(
        _body,
        out_type=jax.ShapeDtypeStruct((N, D), jnp.float32),
        mesh=mesh,
        compiler_params=pltpu.CompilerParams(needs_layout_passes=False),
        scratch_types=(
            ([pltpu.VMEM((CHUNK,), jnp.int32)] * 5
             + [pltpu.VMEM((CHUNK,), jnp.float32)]) * 2
            + [pltpu.VMEM((CHUNK, D), jnp.float32)] * 5
            + [pltpu.VMEM((MAX_DEPTH, D), jnp.float32),
               pltpu.VMEM((MAX_SIBLING, D), jnp.float32),
               pltpu.VMEM((N_NODE_TYPES, D), jnp.float32),
               pltpu.VMEM((D,), jnp.float32),
               pltpu.VMEM((D,), jnp.float32)]
            + [pltpu.SemaphoreType.DMA] * 5
        ),
    )
    return fn(tok, nt, dep, sib, par, key_table, value_table, depth_table,
              sibling_table, node_type_table, parent_key_table,
              ln_gamma, ln_beta)


def kernel(token_ids, node_types, depths, sibling_indices, parent_key_ids,
           key_table, value_table, depth_table, sibling_table,
           node_type_table, parent_key_table, ln_gamma, ln_beta):
    pad = [(0, CHUNK)]
    tok = jnp.pad(token_ids.reshape(N).astype(jnp.int32), pad)
    nt = jnp.pad(node_types.reshape(N).astype(jnp.int32), pad)
    dep = jnp.pad(depths.reshape(N).astype(jnp.int32), pad)
    sib = jnp.pad(sibling_indices.reshape(N).astype(jnp.int32), pad)
    par = jnp.pad(parent_key_ids.reshape(N).astype(jnp.int32), pad)
    out = _sc_embed(tok, nt, dep, sib, par,
                    key_table, value_table, depth_table, sibling_table,
                    node_type_table, parent_key_table,
                    ln_gamma.astype(jnp.float32), ln_beta.astype(jnp.float32))
    return out.reshape(B, L, D)


# speedup vs baseline: 1.6964x; 1.0855x over previous
import jax
import jax.numpy as jnp
from jax import lax
from jax.experimental import pallas as pl
from jax.experimental.pallas import tpu as pltpu
from jax.experimental.pallas import tpu_sc as plsc

NC, NS, LANES = 2, 16, 16
NW = NC * NS

B, L, D = 1024, 200, 128
N = B * L
NTOK_PER_W = N // NW
CHUNK = 80
NCHUNK = NTOK_PER_W // CHUNK
NPAIR = NCHUNK // 2
NGROUP = CHUNK // LANES
NJ = D // LANES
UNROLL = 4

MAX_DEPTH, MAX_SIBLING, N_NODE_TYPES = 64, 256, 4

_INV_D = 1.0 / D
_EPS = 1e-5


def _rsqrt(v):
    bits = plsc.bitcast(v, jnp.int32)
    seed = jnp.full((LANES,), 0x5F3759DF, jnp.int32) - (bits >> 1)
    y = plsc.bitcast(seed, jnp.float32)
    half_v = v * 0.5
    for _ in range(3):
        y = y * (1.5 - half_v * y * y)
    return y


def _body(tok_hbm, nt_hbm, dep_hbm, sib_hbm, par_hbm,
          key_hbm, val_hbm, dept_hbm, sibt_hbm, ntt_hbm, part_hbm,
          gam_hbm, bet_hbm, out_hbm,
          tok_a, nt_a, dep_a, sib_a, par_a, isk_a,
          tok_b, nt_b, dep_b, sib_b, par_b, isk_b,
          keyr0, valr0, parr0, keyr1, valr1, parr1, xbuf0, xbuf1,
          dep_t, sib_t, nt_t, gam_v, bet_v,
          sem_ia, sem_ib, sem_g, sem_o0, sem_o1):
    wid = lax.axis_index("s") * NC + lax.axis_index("c")
    wbase = wid * NTOK_PER_W

    pltpu.sync_copy(dept_hbm, dep_t)
    pltpu.sync_copy(sibt_hbm, sib_t)
    pltpu.sync_copy(ntt_hbm, nt_t)
    pltpu.sync_copy(gam_hbm, gam_v)
    pltpu.sync_copy(bet_hbm, bet_v)

    iota = lax.iota(jnp.int32, LANES)
    colv = [iota + (LANES * j) for j in range(NJ)]
    gms = [gam_v[pl.ds(LANES * j, LANES)] for j in range(NJ)]
    bts = [bet_v[pl.ds(LANES * j, LANES)] for j in range(NJ)]

    idx_srcs = (tok_hbm, nt_hbm, dep_hbm, sib_hbm, par_hbm)
    idx_a = (tok_a, nt_a, dep_a, sib_a, par_a)
    idx_b = (tok_b, nt_b, dep_b, sib_b, par_b)

    def issue_idx(bufs, sem, base):
        return [pltpu.async_copy(src.at[pl.ds(base, CHUNK)], buf, sem)
                for src, buf in zip(idx_srcs, bufs)]

    def drain_idx(bufs, sem):
        for src, buf in zip(idx_srcs, bufs):
            pltpu.make_async_copy(src.at[pl.ds(0, CHUNK)], buf, sem).wait()

    rowsets = [(keyr0, valr0, parr0), (keyr1, valr1, parr1)]

    def issue_gathers(rset, tok_i, par_i):
        kb, vb, pb = rowsets[rset]
        return [pltpu.async_copy(key_hbm.at[tok_i], kb, sem_g),
                pltpu.async_copy(val_hbm.at[tok_i], vb, sem_g),
                pltpu.async_copy(part_hbm.at[par_i], pb, sem_g)]

    def build_mask(nt_i, isk_f):
        for g in range(NGROUP):
            gsl = pl.ds(g * LANES, LANES)
            ntg = nt_i[gsl]
            isk_f[gsl] = jnp.where((ntg == 0) | (ntg == 2), 1.0, 0.0)

    def compute(rset, dep_i, sib_i, nt_i, isk_f, xb):
        keyrows, valrows, parrows = rowsets[rset]

        @plsc.parallel_loop(0, CHUNK, 1, unroll=UNROLL)
        def token_body(t):
            tsp = jnp.full((LANES,), t, jnp.int32)
            m = plsc.load_gather(isk_f, [tsp])
            dsp = plsc.load_gather(dep_i, [tsp])
            ssp = plsc.load_gather(sib_i, [tsp])
            nsp = plsc.load_gather(nt_i, [tsp])
            xs = []
            acs = jnp.zeros((LANES,), jnp.float32)
            acq = jnp.zeros((LANES,), jnp.float32)
            for j in range(NJ):
                kd = plsc.load_gather(keyrows, [tsp, colv[j]])
                vd = plsc.load_gather(valrows, [tsp, colv[j]])
                pd = plsc.load_gather(parrows, [tsp, colv[j]])
                dd = plsc.load_gather(dep_t, [dsp, colv[j]])
                sd = plsc.load_gather(sib_t, [ssp, colv[j]])
                nd = plsc.load_gather(nt_t, [nsp, colv[j]])
                x = vd + (kd - vd) * m + pd + dd + sd + nd
                xs.append(x)
                acs = acs + x
                acq = acq + x * x
            mu = jnp.full((LANES,), jnp.sum(acs)) * _INV_D
            msq = jnp.full((LANES,), jnp.sum(acq)) * _INV_D
            a = _rsqrt(msq - mu * mu + _EPS)
            b = -mu * a
            for j in range(NJ):
                y = (xs[j] * a + b) * gms[j] + bts[j]
                plsc.store_scatter(xb, [tsp, colv[j]], y)

    def drain_out(xb, so):
        pltpu.make_async_copy(xb, out_hbm.at[pl.ds(wbase, CHUNK)], so).wait()

    pltpu.async_copy(
        xbuf0, out_hbm.at[pl.ds(wbase + NTOK_PER_W - 2 * CHUNK, CHUNK)],
        sem_o0)
    pltpu.async_copy(
        xbuf1, out_hbm.at[pl.ds(wbase + NTOK_PER_W - CHUNK, CHUNK)],
        sem_o1)
    issue_idx(idx_a, sem_ia, wbase)

    def pair_body(bp, carry):
        base_a = wbase + bp * (2 * CHUNK)
        base_b = base_a + CHUNK

        drain_idx(idx_a, sem_ia)
        ga = issue_gathers(0, tok_a, par_a)
        ib = issue_idx(idx_b, sem_ib, base_b)
        build_mask(nt_a, isk_a)
        for dsc in ga:
            dsc.wait()
        drain_out(xbuf0, sem_o0)
        for dsc in ib:
            dsc.wait()
        gb = issue_gathers(1, tok_b, par_b)
        compute(0, dep_a, sib_a, nt_a, isk_a, xbuf0)
        pltpu.async_copy(xbuf0, out_hbm.at[pl.ds(base_a, CHUNK)], sem_o0)

        issue_idx(idx_a, sem_ia, base_b + CHUNK)
        build_mask(nt_b, isk_b)
        for dsc in gb:
            dsc.wait()
        drain_out(xbuf1, sem_o1)
        compute(1, dep_b, sib_b, nt_b, isk_b, xbuf1)
        pltpu.async_copy(xbuf1, out_hbm.at[pl.ds(base_b, CHUNK)], sem_o1)
        return carry

    lax.fori_loop(0, NPAIR, pair_body, 0)

    drain_idx(idx_a, sem_ia)
    drain_out(xbuf0, sem_o0)
    drain_out(xbuf1, sem_o1)


@jax.jit
def _sc_embed(tok, nt, dep, sib, par, key_table, value_table, depth_table,
              sibling_table, node_type_table, parent_key_table,
              ln_gamma, ln_beta):
    mesh = plsc.VectorSubcoreMesh(core_axis_name="c", subcore_axis_name="s",
                                  num_cores=NC, num_subcores=NS)
    fn = pl.kernel(
        _body,
        out_type=jax.ShapeDtypeStruct((N, D), jnp.float32),
        mesh=mesh,
        compiler_params=pltpu.CompilerParams(needs_layout_passes=False),
        scratch_types=(
            ([pltpu.VMEM((CHUNK,), jnp.int32)] * 5
             + [pltpu.VMEM((CHUNK,), jnp.float32)]) * 2
            + [pltpu.VMEM((CHUNK, D), jnp.float32)] * 8
            + [pltpu.VMEM((MAX_DEPTH, D), jnp.float32),
               pltpu.VMEM((MAX_SIBLING, D), jnp.float32),
               pltpu.VMEM((N_NODE_TYPES, D), jnp.float32),
               pltpu.VMEM((D,), jnp.float32),
               pltpu.VMEM((D,), jnp.float32)]
            + [pltpu.SemaphoreType.DMA] * 5
        ),
    )
    return fn(tok, nt, dep, sib, par, key_table, value_table, depth_table,
              sibling_table, node_type_table, parent_key_table,
              ln_gamma, ln_beta)


def kernel(token_ids, node_types, depths, sibling_indices, parent_key_ids,
           key_table, value_table, depth_table, sibling_table,
           node_type_table, parent_key_table, ln_gamma, ln_beta):
    pad = [(0, CHUNK)]
    tok = jnp.pad(token_ids.reshape(N).astype(jnp.int32), pad)
    nt = jnp.pad(node_types.reshape(N).astype(jnp.int32), pad)
    dep = jnp.pad(depths.reshape(N).astype(jnp.int32), pad)
    sib = jnp.pad(sibling_indices.reshape(N).astype(jnp.int32), pad)
    par = jnp.pad(parent_key_ids.reshape(N).astype(jnp.int32), pad)
    out = _sc_embed(tok, nt, dep, sib, par,
                    key_table, value_table, depth_table, sibling_table,
                    node_type_table, parent_key_table,
                    ln_gamma.astype(jnp.float32), ln_beta.astype(jnp.float32))
    return out.reshape(B, L, D)


# speedup vs baseline: 1.7946x; 1.0579x over previous
import jax
import jax.numpy as jnp
from jax import lax
from jax.experimental import pallas as pl
from jax.experimental.pallas import tpu as pltpu
from jax.experimental.pallas import tpu_sc as plsc

NC, NS, LANES = 2, 16, 16
NW = NC * NS

B, L, D = 1024, 200, 128
N = B * L
NTOK_PER_W = N // NW
CHUNK = 80
NCHUNK = NTOK_PER_W // CHUNK
NPAIR = NCHUNK // 2
NGROUP = CHUNK // LANES
NJ = D // LANES
UNROLL = 4

MAX_DEPTH, MAX_SIBLING, N_NODE_TYPES = 64, 256, 4

_INV_D = 1.0 / D
_EPS = 1e-5


def _rsqrt(v):
    bits = plsc.bitcast(v, jnp.int32)
    seed = jnp.full((LANES,), 0x5F3759DF, jnp.int32) - (bits >> 1)
    y = plsc.bitcast(seed, jnp.float32)
    half_v = v * 0.5
    for _ in range(3):
        y = y * (1.5 - half_v * y * y)
    return y


def _body(tok_hbm, nt_hbm, dep_hbm, sib_hbm, par_hbm,
          key_hbm, val_hbm, dept_hbm, sibt_hbm, ntt_hbm, part_hbm,
          gam_hbm, bet_hbm, out_hbm,
          tok_a, nt_a, dep_a, sib_a, par_a, isk_a,
          tok_b, nt_b, dep_b, sib_b, par_b, isk_b,
          keyr0, valr0, parr0, keyr1, valr1, parr1, xbuf0, xbuf1,
          dep_t, sib_t, nt_t, gam_v, bet_v,
          sem_ia, sem_ib, sem_g, sem_o0, sem_o1):
    wid = lax.axis_index("s") * NC + lax.axis_index("c")
    wbase = wid * NTOK_PER_W

    pltpu.sync_copy(dept_hbm, dep_t)
    pltpu.sync_copy(sibt_hbm, sib_t)
    pltpu.sync_copy(ntt_hbm, nt_t)
    pltpu.sync_copy(gam_hbm, gam_v)
    pltpu.sync_copy(bet_hbm, bet_v)

    iota = lax.iota(jnp.int32, LANES)
    colv = [iota + (LANES * j) for j in range(NJ)]
    gms = [gam_v[pl.ds(LANES * j, LANES)] for j in range(NJ)]
    bts = [bet_v[pl.ds(LANES * j, LANES)] for j in range(NJ)]

    idx_srcs = (tok_hbm, nt_hbm, dep_hbm, sib_hbm, par_hbm)
    idx_a = (tok_a, nt_a, dep_a, sib_a, par_a)
    idx_b = (tok_b, nt_b, dep_b, sib_b, par_b)

    def issue_idx(bufs, sem, base):
        return [pltpu.async_copy(src.at[pl.ds(base, CHUNK)], buf, sem)
                for src, buf in zip(idx_srcs, bufs)]

    def drain_idx(bufs, sem):
        for src, buf in zip(idx_srcs, bufs):
            pltpu.make_async_copy(src.at[pl.ds(0, CHUNK)], buf, sem).wait()

    rowsets = [(keyr0, valr0, parr0), (keyr1, valr1, parr1)]

    def issue_gathers(rset, tok_i, par_i):
        kb, vb, pb = rowsets[rset]
        return [pltpu.async_copy(key_hbm.at[tok_i], kb, sem_g),
                pltpu.async_copy(val_hbm.at[tok_i], vb, sem_g),
                pltpu.async_copy(part_hbm.at[par_i], pb, sem_g)]

    def build_mask(nt_i, isk_f):
        for g in range(NGROUP):
            gsl = pl.ds(g * LANES, LANES)
            ntg = nt_i[gsl]
            isk_f[gsl] = jnp.where((ntg == 0) | (ntg == 2), 1.0, 0.0)

    def compute(rset, dep_i, sib_i, nt_i, isk_f, xb):
        keyrows, valrows, parrows = rowsets[rset]

        @plsc.parallel_loop(0, CHUNK, 1, unroll=UNROLL)
        def token_body(t):
            tsp = jnp.full((LANES,), t, jnp.int32)
            m = plsc.load_gather(isk_f, [tsp])
            dsp = plsc.load_gather(dep_i, [tsp])
            ssp = plsc.load_gather(sib_i, [tsp])
            nsp = plsc.load_gather(nt_i, [tsp])
            xs = []
            acs = jnp.zeros((LANES,), jnp.float32)
            acq = jnp.zeros((LANES,), jnp.float32)
            for j in range(NJ):
                kd = plsc.load_gather(keyrows, [tsp, colv[j]])
                vd = plsc.load_gather(valrows, [tsp, colv[j]])
                pd = plsc.load_gather(parrows, [tsp, colv[j]])
                dd = plsc.load_gather(dep_t, [dsp, colv[j]])
                sd = plsc.load_gather(sib_t, [ssp, colv[j]])
                nd = plsc.load_gather(nt_t, [nsp, colv[j]])
                x = vd + (kd - vd) * m + pd + dd + sd + nd
                xs.append(x)
                acs = acs + x
                acq = acq + x * x
            mu = jnp.full((LANES,), jnp.sum(acs)) * _INV_D
            msq = jnp.full((LANES,), jnp.sum(acq)) * _INV_D
            a = _rsqrt(msq - mu * mu + _EPS)
            b = -mu * a
            for j in range(NJ):
                y = (xs[j] * a + b) * gms[j] + bts[j]
                plsc.store_scatter(xb, [tsp, colv[j]], y)

    def drain_out(xb, so):
        pltpu.make_async_copy(xb, out_hbm.at[pl.ds(wbase, CHUNK)], so).wait()

    def drain_gathers(rset):
        kb, vb, pb = rowsets[rset]
        pltpu.make_async_copy(key_hbm.at[pl.ds(0, CHUNK)], kb, sem_g).wait()
        pltpu.make_async_copy(val_hbm.at[pl.ds(0, CHUNK)], vb, sem_g).wait()
        pltpu.make_async_copy(part_hbm.at[pl.ds(0, CHUNK)], pb, sem_g).wait()

    pltpu.async_copy(
        xbuf0, out_hbm.at[pl.ds(wbase + NTOK_PER_W - 2 * CHUNK, CHUNK)],
        sem_o0)
    pltpu.async_copy(
        xbuf1, out_hbm.at[pl.ds(wbase + NTOK_PER_W - CHUNK, CHUNK)],
        sem_o1)
    for dsc in issue_idx(idx_a, sem_ia, wbase):
        dsc.wait()
    issue_gathers(0, tok_a, par_a)

    def pair_body(bp, carry):
        base_a = wbase + bp * (2 * CHUNK)
        base_b = base_a + CHUNK

        ib = issue_idx(idx_b, sem_ib, base_b)
        build_mask(nt_a, isk_a)
        drain_gathers(0)
        drain_out(xbuf0, sem_o0)
        for dsc in ib:
            dsc.wait()
        gb = issue_gathers(1, tok_b, par_b)
        compute(0, dep_a, sib_a, nt_a, isk_a, xbuf0)
        pltpu.async_copy(xbuf0, out_hbm.at[pl.ds(base_a, CHUNK)], sem_o0)

        ia = issue_idx(idx_a, sem_ia, base_b + CHUNK)
        build_mask(nt_b, isk_b)
        for dsc in gb:
            dsc.wait()
        drain_out(xbuf1, sem_o1)
        for dsc in ia:
            dsc.wait()
        issue_gathers(0, tok_a, par_a)
        compute(1, dep_b, sib_b, nt_b, isk_b, xbuf1)
        pltpu.async_copy(xbuf1, out_hbm.at[pl.ds(base_b, CHUNK)], sem_o1)
        return carry

    lax.fori_loop(0, NPAIR, pair_body, 0)

    drain_gathers(0)
    drain_out(xbuf0, sem_o0)
    drain_out(xbuf1, sem_o1)


@jax.jit
def _sc_embed(tok, nt, dep, sib, par, key_table, value_table, depth_table,
              sibling_table, node_type_table, parent_key_table,
              ln_gamma, ln_beta):
    mesh = plsc.VectorSubcoreMesh(core_axis_name="c", subcore_axis_name="s",
                                  num_cores=NC, num_subcores=NS)
    fn = pl.kernel(
        _body,
        out_type=jax.ShapeDtypeStruct((N, D), jnp.float32),
        mesh=mesh,
        compiler_params=pltpu.CompilerParams(needs_layout_passes=False),
        scratch_types=(
            ([pltpu.VMEM((CHUNK,), jnp.int32)] * 5
             + [pltpu.VMEM((CHUNK,), jnp.float32)]) * 2
            + [pltpu.VMEM((CHUNK, D), jnp.float32)] * 8
            + [pltpu.VMEM((MAX_DEPTH, D), jnp.float32),
               pltpu.VMEM((MAX_SIBLING, D), jnp.float32),
               pltpu.VMEM((N_NODE_TYPES, D), jnp.float32),
               pltpu.VMEM((D,), jnp.float32),
               pltpu.VMEM((D,), jnp.float32)]
            + [pltpu.SemaphoreType.DMA] * 5
        ),
    )
    return fn(tok, nt, dep, sib, par, key_table, value_table, depth_table,
              sibling_table, node_type_table, parent_key_table,
              ln_gamma, ln_beta)


def kernel(token_ids, node_types, depths, sibling_indices, parent_key_ids,
           key_table, value_table, depth_table, sibling_table,
           node_type_table, parent_key_table, ln_gamma, ln_beta):
    pad = [(0, CHUNK)]
    tok = jnp.pad(token_ids.reshape(N).astype(jnp.int32), pad)
    nt = jnp.pad(node_types.reshape(N).astype(jnp.int32), pad)
    dep = jnp.pad(depths.reshape(N).astype(jnp.int32), pad)
    sib = jnp.pad(sibling_indices.reshape(N).astype(jnp.int32), pad)
    par = jnp.pad(parent_key_ids.reshape(N).astype(jnp.int32), pad)
    out = _sc_embed(tok, nt, dep, sib, par,
                    key_table, value_table, depth_table, sibling_table,
                    node_type_table, parent_key_table,
                    ln_gamma.astype(jnp.float32), ln_beta.astype(jnp.float32))
    return out.reshape(B, L, D)
